# Initial kernel scaffold; baseline (speedup 1.0000x reference)
import jax
import jax.numpy as jnp
from jax.experimental import pallas as pl


def kernel(enc_h_n, enc_c_n, enc_outputs, enc_inputs, Emb, W_ih, W_hh, b_lstm, W_out, b_out):
    raise NotImplementedError("write your pallas kernel here")


# speedup vs baseline: 9.0600x; 9.0600x over previous
import jax
import jax.numpy as jnp
from jax.experimental import pallas as pl
from jax.experimental.pallas import tpu as pltpu

_BEAM = 4
_START = 1
_VOCAB = 8000
_VPAD = 8192
_HIDDEN = 512
_EMB = 256
_SEQ = 12
_B = 32
_BB = _B * _BEAM


def _softmax_rows(x):
    m = jnp.max(x, axis=1, keepdims=True)
    e = jnp.exp(x - m)
    return e / jnp.sum(e, axis=1, keepdims=True)


def _decoder_step(emb, h0, c0, enc_t, enc_in, Wih, Whh, bl, Wout, bo):
    dn_s = (((2,), (1,)), ((1,), (0,)))
    scores = jax.lax.dot_general(
        enc_t, h0.astype(jnp.bfloat16), dn_s,
        preferred_element_type=jnp.float32).T
    smax = jnp.max(scores, axis=0, keepdims=True)
    se = jnp.exp(scores - smax)
    attn = se / jnp.sum(se, axis=0, keepdims=True)
    dn_c = (((0,), (0,)), ((1,), (1,)))
    ctx = jax.lax.dot_general(
        attn.astype(jnp.bfloat16), enc_t, dn_c,
        preferred_element_type=jnp.float32)
    x = jnp.concatenate([emb, ctx, enc_in], axis=1)
    ones_g = (bl - bl) + 1.0
    gA = jnp.dot(x.astype(jnp.bfloat16), Wih,
                 preferred_element_type=jnp.float32) * ones_g[None, :]
    gB = jnp.dot(h0.astype(jnp.bfloat16), Whh,
                 preferred_element_type=jnp.float32) * ones_g[None, :]
    gates = (gA + gB) + bl[None, :]
    i = jax.nn.sigmoid(gates[:, 0 * _HIDDEN:1 * _HIDDEN])
    f = jax.nn.sigmoid(gates[:, 1 * _HIDDEN:2 * _HIDDEN])
    g = jnp.tanh(gates[:, 2 * _HIDDEN:3 * _HIDDEN])
    o = jax.nn.sigmoid(gates[:, 3 * _HIDDEN:4 * _HIDDEN])
    c = f * c0 + i * g
    h = o * jnp.tanh(c)
    ones_l = (bo - bo) + 1.0
    logits = jnp.dot(h.astype(jnp.bfloat16), Wout,
                     preferred_element_type=jnp.float32)[:, :_VOCAB]
    logits = logits * ones_l[None, :] + bo[None, :]
    return _softmax_rows(logits), h, c


def _topk4(vals, lanes, idx_base):
    out_v, out_i = [], []
    for _ in range(_BEAM):
        m = jnp.max(vals, axis=1, keepdims=True)
        idx = jnp.min(jnp.where(vals == m, lanes, jnp.int32(_VOCAB)),
                      axis=1, keepdims=True)
        out_v.append(m)
        out_i.append(idx + idx_base)
        vals = jnp.where(lanes == idx, jnp.float32(-1.0), vals)
    return out_v, out_i


def _beam_kernel(enc_t_ref, enc_in_ref, emb0_ref, h0_ref, c0_ref, Emb_ref,
                 Wih_ref, Whh_ref, bl_ref, Wout_ref, bo_ref, out_ref):
    Wih = Wih_ref[:]
    Whh = Whh_ref[:]
    bl = bl_ref[:]
    Wout = Wout_ref[:]
    bo = bo_ref[:]
    enc_t = enc_t_ref[:]
    Emb = Emb_ref[:]

    lanes_b = jax.lax.broadcasted_iota(jnp.int32, (_B, _VOCAB), 1)
    lanes_bb = jax.lax.broadcasted_iota(jnp.int32, (_BB, _VOCAB), 1)

    p, h, c = _decoder_step(emb0_ref[:], h0_ref[:], c0_ref[:],
                            enc_t[:, :_B, :], enc_in_ref[0, :_B, :],
                            Wih, Whh, bl, Wout, bo)
    init_v, init_i = _topk4(p, lanes_b, jnp.int32(0))
    scores = init_v
    out_ref[0] = jnp.concatenate(init_i, axis=1)
    words_col = jnp.concatenate(init_i, axis=0)
    h = jnp.concatenate([h] * _BEAM, axis=0)
    c = jnp.concatenate([c] * _BEAM, axis=0)

    for step in range(1, _SEQ):
        onehot = (lanes_bb == words_col).astype(jnp.bfloat16)
        emb = jnp.dot(onehot, Emb, preferred_element_type=jnp.float32)
        p, h, c = _decoder_step(emb, h, c, enc_t, enc_in_ref[step],
                                Wih, Whh, bl, Wout, bo)
        cand_v, cand_i = [], []
        for b in range(_BEAM):
            pb = p[b * _B:(b + 1) * _B, :] * scores[b]
            vs, ix = _topk4(pb, lanes_b, jnp.int32(b * _VOCAB))
            cand_v += vs
            cand_i += ix
        cv = jnp.concatenate(cand_v, axis=1)
        ci = jnp.concatenate(cand_i, axis=1)
        sel = []
        for _ in range(_BEAM):
            m = jnp.max(cv, axis=1, keepdims=True)
            gi = jnp.min(jnp.where(cv == m, ci, jnp.int32(_BEAM * _VOCAB)),
                         axis=1, keepdims=True)
            sel.append(gi)
            cv = jnp.where(ci == gi, jnp.float32(-1.0), cv)
        out_ref[step] = jnp.concatenate(sel, axis=1)
        words_col = jnp.concatenate([jnp.mod(gi, _VOCAB) for gi in sel],
                                    axis=0)


def kernel(enc_h_n, enc_c_n, enc_outputs, enc_inputs, Emb, W_ih, W_hh,
           b_lstm, W_out, b_out):
    enc_t = jnp.concatenate([enc_outputs] * _BEAM, axis=1).astype(jnp.bfloat16)
    enc_in_t = jnp.concatenate([enc_inputs] * _BEAM, axis=1)
    emb0 = jnp.broadcast_to(Emb[_START], (_B, _EMB))
    h0 = enc_h_n[0]
    c0 = enc_c_n[0]
    Wih_t = W_ih.T.astype(jnp.bfloat16)
    Whh_t = W_hh.T.astype(jnp.bfloat16)
    Wout_t = jnp.zeros((_HIDDEN, _VPAD), jnp.bfloat16).at[:, :_VOCAB].set(
        W_out.T.astype(jnp.bfloat16))
    Emb_b = Emb.astype(jnp.bfloat16)

    out = pl.pallas_call(
        _beam_kernel,
        out_shape=jax.ShapeDtypeStruct((_SEQ, _B, _BEAM), jnp.int32),
        compiler_params=pltpu.CompilerParams(
            vmem_limit_bytes=128 * 1024 * 1024),
    )(enc_t, enc_in_t, emb0, h0, c0, Emb_b, Wih_t, Whh_t, b_lstm, Wout_t,
      b_out)
    return jnp.transpose(out, (2, 1, 0))
